# baseline (speedup 1.0000x reference)
import functools

import numpy as np

import jax
import jax.numpy as jnp
from jax import lax
from jax.experimental import pallas as pl
from jax.experimental.pallas import tpu as pltpu
from jax.experimental.pallas import tpu_sc as plsc

VOCAB = 1000000
DIM = 64
B = 4096
L = 200

NC = 2
NS = 16
NW = NC * NS
IPW = B // NW
HALF = L // 2

_WPERM = np.concatenate(
    [np.arange(32 * k, 32 * (k + 1)).reshape(16, 2).T.reshape(32) for k in range(4)]
)

_mesh = plsc.VectorSubcoreMesh(core_axis_name="c", subcore_axis_name="s")


@functools.partial(
    pl.kernel,
    out_type=jax.ShapeDtypeStruct((B,), jnp.float32),
    mesh=_mesh,
    scratch_types=[
        pltpu.VMEM((4 * IPW, HALF), jnp.int32),
        pltpu.VMEM((2 * L, DIM), jnp.bfloat16),
        pltpu.VMEM((2 * L, DIM), jnp.bfloat16),
        pltpu.VMEM((2 * DIM + 16,), jnp.float32),
        pltpu.VMEM((IPW,), jnp.float32),
        pltpu.SemaphoreType.DMA,
        pltpu.SemaphoreType.DMA,
    ],
    compiler_params=pltpu.CompilerParams(
        use_tc_tiling_on_sc=False, needs_layout_passes=False),
)
def _sc_kernel(idx_hbm, table_hbm, wb_hbm, out_hbm,
               idx_v, buf0, buf1, wb_v, out_v, sem0, sem1):
    wid = lax.axis_index("s") * NC + lax.axis_index("c")
    row0 = wid * (4 * IPW)

    pltpu.sync_copy(idx_hbm.at[pl.ds(row0, 4 * IPW)], idx_v)
    pltpu.sync_copy(wb_hbm, wb_v)

    lanes = lax.iota(jnp.int32, 16)

    def fire(item, buf, sem):
        for r in range(4):
            pltpu.async_copy(
                table_hbm.at[idx_v.at[4 * item + r]],
                buf.at[pl.ds(HALF * r, HALF)],
                sem,
            )

    def drain(buf, sem):
        pltpu.make_async_copy(table_hbm.at[pl.ds(0, 2 * L)], buf, sem).wait()

    def widen(acc):
        return plsc.unpack(acc, format=plsc.PackFormat.INTERLEAVED)

    def process(buf, item):
        neg = jnp.full((32,), -jnp.inf, jnp.bfloat16)

        def jbody(j, carry):
            out = []
            for d in range(2):
                out.append(jnp.maximum(carry[d], buf[j, pl.ds(32 * d, 32)]))
            for d in range(2):
                out.append(jnp.maximum(carry[2 + d], buf[L + j, pl.ds(32 * d, 32)]))
            return tuple(out)

        acc = lax.fori_loop(0, L, jbody, (neg,) * 4)

        z = jnp.zeros((16,), jnp.float32)
        for k in range(4):
            even, odd = widen(acc[k])
            z = z + even * wb_v[pl.ds(32 * k, 16)]
            z = z + odd * wb_v[pl.ds(32 * k + 16, 16)]
        dnums = lax.GatherDimensionNumbers(
            offset_dims=(), collapsed_slice_dims=(0,), start_index_map=(0,))
        for k in (8, 4, 2, 1):
            shuf = lax.gather(
                z, (lanes ^ k).reshape(16, 1), dnums, (1,),
                mode=lax.GatherScatterMode.PROMISE_IN_BOUNDS)
            z = z + shuf
        logits = wb_v[pl.ds(2 * DIM, 16)] + z
        return 1.0 / (1.0 + jnp.exp(-logits))

    fire(0, buf0, sem0)

    def cbody(i, accv):
        c = 2 * i
        fire((c + 1) & (IPW - 1), buf1, sem1)
        drain(buf0, sem0)
        sig0 = process(buf0, c)
        accv = jnp.where(lanes == (c & 15), sig0, accv)
        fire((c + 2) & (IPW - 1), buf0, sem0)
        drain(buf1, sem1)
        sig1 = process(buf1, c + 1)
        accv = jnp.where(lanes == ((c + 1) & 15), sig1, accv)

        @pl.when(((c + 1) & 15) == 15)
        def _():
            out_v[pl.ds(c - 14, 16)] = accv

        return accv

    lax.fori_loop(0, IPW // 2, cbody, jnp.zeros((16,), jnp.float32))

    drain(buf0, sem0)

    pltpu.sync_copy(out_v, out_hbm.at[pl.ds(wid * IPW, IPW)])


def kernel(premise, hypothesis, table, W, b):
    idx = jnp.concatenate(
        [premise.reshape(B, 2, HALF), hypothesis.reshape(B, 2, HALF)], axis=1
    ).reshape(4 * B, HALF)
    table_bf = table.astype(jnp.bfloat16)
    wb = jnp.concatenate(
        [W.reshape(2 * DIM)[_WPERM], jnp.broadcast_to(b, (16,))])
    return _sc_kernel(idx, table_bf, wb)


# speedup vs baseline: 1.1923x; 1.1923x over previous
import functools

import jax
import jax.numpy as jnp
from jax import lax
from jax.experimental import pallas as pl
from jax.experimental.pallas import tpu as pltpu
from jax.experimental.pallas import tpu_sc as plsc

VOCAB = 1000000
DIM = 64
B = 4096
L = 200

NC = 2
NS = 16
NW = NC * NS
IPW = B // NW
HALF = L // 2

_mesh = plsc.VectorSubcoreMesh(core_axis_name="c", subcore_axis_name="s")


@functools.partial(
    pl.kernel,
    out_type=jax.ShapeDtypeStruct((B,), jnp.float32),
    mesh=_mesh,
    scratch_types=[
        pltpu.VMEM((4 * IPW, HALF), jnp.int32),
        pltpu.VMEM((2 * L, DIM), jnp.float32),
        pltpu.VMEM((2 * L, DIM), jnp.float32),
        pltpu.VMEM((2 * DIM + 16,), jnp.float32),
        pltpu.VMEM((IPW,), jnp.float32),
        pltpu.SemaphoreType.DMA,
        pltpu.SemaphoreType.DMA,
    ],
    compiler_params=pltpu.CompilerParams(
        use_tc_tiling_on_sc=False, needs_layout_passes=False),
)
def _sc_kernel(idx_hbm, table_hbm, wb_hbm, out_hbm,
               idx_v, buf0, buf1, wb_v, out_v, sem0, sem1):
    wid = lax.axis_index("s") * NC + lax.axis_index("c")
    row0 = wid * (4 * IPW)

    pltpu.sync_copy(idx_hbm.at[pl.ds(row0, 4 * IPW)], idx_v)
    pltpu.sync_copy(wb_hbm, wb_v)

    lanes = lax.iota(jnp.int32, 16)

    def fire(item, buf, sem):
        for r in range(4):
            pltpu.async_copy(
                table_hbm.at[idx_v.at[4 * item + r]],
                buf.at[pl.ds(HALF * r, HALF)],
                sem,
            )

    def drain(buf, sem):
        pltpu.make_async_copy(table_hbm.at[pl.ds(0, 2 * L)], buf, sem).wait()

    def process(buf, item):
        neg = jnp.full((16,), -jnp.inf, jnp.float32)

        def jbody(j, carry):
            out = []
            for d in range(4):
                out.append(jnp.maximum(carry[d], buf[j, pl.ds(16 * d, 16)]))
            for d in range(4):
                out.append(jnp.maximum(carry[4 + d], buf[L + j, pl.ds(16 * d, 16)]))
            return tuple(out)

        acc = lax.fori_loop(0, L, jbody, (neg,) * 8, unroll=2)

        z = jnp.zeros((16,), jnp.float32)
        for d in range(4):
            z = z + acc[d] * wb_v[pl.ds(16 * d, 16)]
        for d in range(4):
            z = z + acc[4 + d] * wb_v[pl.ds(DIM + 16 * d, 16)]
        dnums = lax.GatherDimensionNumbers(
            offset_dims=(), collapsed_slice_dims=(0,), start_index_map=(0,))
        for k in (8, 4, 2, 1):
            shuf = lax.gather(
                z, (lanes ^ k).reshape(16, 1), dnums, (1,),
                mode=lax.GatherScatterMode.PROMISE_IN_BOUNDS)
            z = z + shuf
        logits = wb_v[pl.ds(2 * DIM, 16)] + z
        return 1.0 / (1.0 + jnp.exp(-logits))

    fire(0, buf0, sem0)

    def cbody(i, accv):
        c = 2 * i
        fire((c + 1) & (IPW - 1), buf1, sem1)
        drain(buf0, sem0)
        sig0 = process(buf0, c)
        accv = jnp.where(lanes == (c & 15), sig0, accv)
        fire((c + 2) & (IPW - 1), buf0, sem0)
        drain(buf1, sem1)
        sig1 = process(buf1, c + 1)
        accv = jnp.where(lanes == ((c + 1) & 15), sig1, accv)

        @pl.when(((c + 1) & 15) == 15)
        def _():
            out_v[pl.ds(c - 14, 16)] = accv

        return accv

    lax.fori_loop(0, IPW // 2, cbody, jnp.zeros((16,), jnp.float32))

    drain(buf0, sem0)

    pltpu.sync_copy(out_v, out_hbm.at[pl.ds(wid * IPW, IPW)])


def kernel(premise, hypothesis, table, W, b):
    idx = jnp.concatenate(
        [premise.reshape(B, 2, HALF), hypothesis.reshape(B, 2, HALF)], axis=1
    ).reshape(4 * B, HALF)
    wb = jnp.concatenate([W.reshape(2 * DIM), jnp.broadcast_to(b, (16,))])
    return _sc_kernel(idx, table, wb)
